# Initial kernel scaffold; baseline (speedup 1.0000x reference)
import jax
import jax.numpy as jnp
from jax.experimental import pallas as pl


def kernel(hour, dow, dom, hour_table, dow_table):
    raise NotImplementedError("write your pallas kernel here")


# speedup vs baseline: 3.4239x; 3.4239x over previous
import jax
import jax.numpy as jnp
from jax import lax
from jax.experimental import pallas as pl
from jax.experimental.pallas import tpu as pltpu
from jax.experimental.pallas import tpu_sc as plsc

B = 16384
DH = 8
DD = 4
D = DH + DD
NC = 2
NS = 16
NW = NC * NS
BPW = B // NW
L = 16


def _emb_body(hour_hbm, dow_hbm, ht_hbm, dt_hbm, out_hbm,
              hour_v, dow_v, ht_v, dt_v, out_v):
    wid = lax.axis_index("s") * NC + lax.axis_index("c")
    base = wid * BPW
    pltpu.sync_copy(ht_hbm, ht_v)
    pltpu.sync_copy(dt_hbm, dt_v)
    pltpu.sync_copy(hour_hbm.at[pl.ds(base, BPW)], hour_v)
    pltpu.sync_copy(dow_hbm.at[pl.ds(base, BPW)], dow_v)
    i12 = lax.iota(jnp.int32, L) * D
    for g in range(BPW // L):
        sl = pl.ds(g * L, L)
        hb = hour_v[sl] * DH
        db = dow_v[sl] * DD
        for c in range(DH):
            vals = plsc.load_gather(ht_v, [hb + c])
            plsc.store_scatter(out_v, [i12 + (g * L * D + c)], vals)
        for c in range(DD):
            vals = plsc.load_gather(dt_v, [db + c])
            plsc.store_scatter(out_v, [i12 + (g * L * D + DH + c)], vals)
    pltpu.sync_copy(out_v, out_hbm.at[pl.ds(base * D, BPW * D)])


@jax.jit
def _lookup(hour, dow, ht_flat, dt_flat):
    mesh = plsc.VectorSubcoreMesh(core_axis_name="c", subcore_axis_name="s")
    f = pl.kernel(
        _emb_body,
        out_type=jax.ShapeDtypeStruct((B * D,), jnp.float32),
        mesh=mesh,
        compiler_params=pltpu.CompilerParams(needs_layout_passes=False),
        scratch_types=[
            pltpu.VMEM((BPW,), jnp.int32),
            pltpu.VMEM((BPW,), jnp.int32),
            pltpu.VMEM((24 * DH,), jnp.float32),
            pltpu.VMEM((7 * DD,), jnp.float32),
            pltpu.VMEM((BPW * D,), jnp.float32),
        ],
    )
    return f(hour, dow, ht_flat, dt_flat)


def kernel(hour, dow, dom, hour_table, dow_table):
    del dom
    flat = _lookup(hour.astype(jnp.int32), dow.astype(jnp.int32),
                   hour_table.reshape(-1), dow_table.reshape(-1))
    return flat.reshape(B, D)


# speedup vs baseline: 3.8654x; 1.1289x over previous
import jax
import jax.numpy as jnp
from jax import lax
from jax.experimental import pallas as pl
from jax.experimental.pallas import tpu as pltpu
from jax.experimental.pallas import tpu_sc as plsc

B = 16384
NH = 24
DH = 8
ND = 7
DD = 4
D = DH + DD
NC = 2
NS = 16
NW = NC * NS
BPW = B // NW
L = 16


def _emb_body(hour_hbm, dow_hbm, ht_hbm, dt_hbm, out_hbm,
              hour_v, dow_v, ht_v, dt_v, out_v, sem):
    wid = lax.axis_index("s") * NC + lax.axis_index("c")
    base = wid * BPW
    copies = [
        pltpu.async_copy(ht_hbm, ht_v, sem),
        pltpu.async_copy(dt_hbm, dt_v, sem),
        pltpu.async_copy(hour_hbm.at[pl.ds(base, BPW)], hour_v, sem),
        pltpu.async_copy(dow_hbm.at[pl.ds(base, BPW)], dow_v, sem),
    ]
    for c in copies:
        c.wait()
    i16 = lax.iota(jnp.int32, L)
    cvec = [jnp.full((L,), c, jnp.int32) for c in range(D)]
    for g in range(BPW // L):
        sl = pl.ds(g * L, L)
        hv = hour_v[sl]
        dv = dow_v[sl]
        rvec = i16 + (g * L)
        for c in range(DH):
            vals = plsc.load_gather(ht_v, [hv, cvec[c]])
            plsc.store_scatter(out_v, [rvec, cvec[c]], vals)
        for c in range(DD):
            vals = plsc.load_gather(dt_v, [dv, cvec[c]])
            plsc.store_scatter(out_v, [rvec, cvec[DH + c]], vals)
    pltpu.sync_copy(out_v, out_hbm.at[pl.ds(base, BPW)])


@jax.jit
def _lookup(hour, dow, ht, dt):
    mesh = plsc.VectorSubcoreMesh(core_axis_name="c", subcore_axis_name="s")
    f = pl.kernel(
        _emb_body,
        out_type=jax.ShapeDtypeStruct((B, D), jnp.float32),
        mesh=mesh,
        compiler_params=pltpu.CompilerParams(needs_layout_passes=False),
        scratch_types=[
            pltpu.VMEM((BPW,), jnp.int32),
            pltpu.VMEM((BPW,), jnp.int32),
            pltpu.VMEM((NH, DH), jnp.float32),
            pltpu.VMEM((ND, DD), jnp.float32),
            pltpu.VMEM((BPW, D), jnp.float32),
            pltpu.SemaphoreType.DMA,
        ],
    )
    return f(hour, dow, ht, dt)


def kernel(hour, dow, dom, hour_table, dow_table):
    del dom
    return _lookup(hour.astype(jnp.int32), dow.astype(jnp.int32),
                   hour_table, dow_table)


# speedup vs baseline: 4.0019x; 1.0353x over previous
import jax
import jax.numpy as jnp
from jax import lax
from jax.experimental import pallas as pl
from jax.experimental.pallas import tpu as pltpu
from jax.experimental.pallas import tpu_sc as plsc

B = 16384
NH = 24
DH = 8
ND = 7
DD = 4
D = DH + DD
NC = 2
NS = 16
NW = NC * NS
BPW = B // NW
L = 16


def _emb_body(hour_hbm, dow_hbm, ht_hbm, dt_hbm, out_hbm,
              hour_v, dow_v, ht_v, dt_v, out_v, sem):
    wid = lax.axis_index("s") * NC + lax.axis_index("c")
    base = wid * BPW
    copies = [
        pltpu.async_copy(ht_hbm, ht_v, sem),
        pltpu.async_copy(dt_hbm, dt_v, sem),
        pltpu.async_copy(hour_hbm.at[pl.ds(base, BPW)], hour_v, sem),
        pltpu.async_copy(dow_hbm.at[pl.ds(base, BPW)], dow_v, sem),
    ]
    for c in copies:
        c.wait()
    i16 = lax.iota(jnp.int32, L)
    cvec = [jnp.full((L,), c, jnp.int32) for c in range(D)]
    for g in range(BPW // L):
        sl = pl.ds(g * L, L)
        hv = hour_v[sl]
        dv = dow_v[sl]
        rvec = i16 + (g * L)
        for c in range(DH):
            vals = plsc.load_gather(ht_v, [hv, cvec[c]])
            plsc.store_scatter(out_v, [rvec, cvec[c]], vals)
        for c in range(DD):
            vals = plsc.load_gather(dt_v, [dv, cvec[c]])
            plsc.store_scatter(out_v, [rvec, cvec[DH + c]], vals)
    pltpu.sync_copy(out_v, out_hbm.at[pl.ds(base, BPW)])


@jax.jit
def _lookup(hour, dow, ht, dt):
    mesh = plsc.VectorSubcoreMesh(core_axis_name="c", subcore_axis_name="s")
    f = pl.kernel(
        _emb_body,
        out_type=jax.ShapeDtypeStruct((B, D), jnp.float32),
        mesh=mesh,
        compiler_params=pltpu.CompilerParams(
            needs_layout_passes=False, use_tc_tiling_on_sc=False),
        scratch_types=[
            pltpu.VMEM((BPW,), jnp.int32),
            pltpu.VMEM((BPW,), jnp.int32),
            pltpu.VMEM((NH, DH), jnp.float32),
            pltpu.VMEM((ND, DD), jnp.float32),
            pltpu.VMEM((BPW, D), jnp.float32),
            pltpu.SemaphoreType.DMA,
        ],
    )
    return f(hour, dow, ht, dt)


def kernel(hour, dow, dom, hour_table, dow_table):
    del dom
    return _lookup(hour.astype(jnp.int32), dow.astype(jnp.int32),
                   hour_table, dow_table)


# speedup vs baseline: 4.1229x; 1.0302x over previous
import jax
import jax.numpy as jnp
from jax import lax
from jax.experimental import pallas as pl
from jax.experimental.pallas import tpu as pltpu
from jax.experimental.pallas import tpu_sc as plsc

B = 16384
NH = 24
DH = 8
ND = 7
DD = 4
D = DH + DD
SH = 17
SD = 5
DBASE = NH * SH
MT = DBASE + ND * SD
MTP = 448
NC = 2
NS = 16
NW = NC * NS
BPW = B // NW
L = 16


def _emb_body(hour_hbm, dow_hbm, mt_hbm, out_hbm,
              hour_v, dow_v, mt_v, out_v, sem):
    wid = lax.axis_index("s") * NC + lax.axis_index("c")
    base = wid * BPW
    copies = [
        pltpu.async_copy(mt_hbm, mt_v, sem),
        pltpu.async_copy(hour_hbm.at[pl.ds(base, BPW)], hour_v, sem),
        pltpu.async_copy(dow_hbm.at[pl.ds(base, BPW)], dow_v, sem),
    ]
    for c in copies:
        c.wait()
    i16 = lax.iota(jnp.int32, L)
    cvec = [jnp.full((L,), c, jnp.int32) for c in range(D)]
    for g in range(BPW // L):
        sl = pl.ds(g * L, L)
        hb = hour_v[sl] * SH
        db = dow_v[sl] * SD + DBASE
        rvec = i16 + (g * L)
        for c in range(DH):
            vals = plsc.load_gather(mt_v, [hb + c])
            plsc.store_scatter(out_v, [rvec, cvec[c]], vals)
        for c in range(DD):
            vals = plsc.load_gather(mt_v, [db + c])
            plsc.store_scatter(out_v, [rvec, cvec[DH + c]], vals)
    pltpu.sync_copy(out_v, out_hbm.at[pl.ds(base, BPW)])


@jax.jit
def _lookup(hour, dow, mt):
    mesh = plsc.VectorSubcoreMesh(core_axis_name="c", subcore_axis_name="s")
    f = pl.kernel(
        _emb_body,
        out_type=jax.ShapeDtypeStruct((B, D), jnp.float32),
        mesh=mesh,
        compiler_params=pltpu.CompilerParams(
            needs_layout_passes=False, use_tc_tiling_on_sc=False),
        scratch_types=[
            pltpu.VMEM((BPW,), jnp.int32),
            pltpu.VMEM((BPW,), jnp.int32),
            pltpu.VMEM((MTP,), jnp.float32),
            pltpu.VMEM((BPW, D), jnp.float32),
            pltpu.SemaphoreType.DMA,
        ],
    )
    return f(hour, dow, mt)


def kernel(hour, dow, dom, hour_table, dow_table):
    del dom
    mt = jnp.concatenate([
        jnp.pad(hour_table, ((0, 0), (0, SH - DH))).reshape(-1),
        jnp.pad(dow_table, ((0, 0), (0, SD - DD))).reshape(-1),
        jnp.zeros((MTP - MT,), jnp.float32),
    ])
    return _lookup(hour.astype(jnp.int32), dow.astype(jnp.int32), mt)


# speedup vs baseline: 5.6789x; 1.3774x over previous
import jax
import jax.numpy as jnp
from jax import lax
from jax.experimental import pallas as pl
from jax.experimental.pallas import tpu as pltpu
from jax.experimental.pallas import tpu_sc as plsc

B = 16384
NH = 24
DH = 8
ND = 7
DD = 4
D = DH + DD
SH = 17
SD = 5
DBASE = NH * SH
MT = DBASE + ND * SD
MTP = 448
NC = 2
NS = 16
NW = NC * NS
BPW = B // NW
L = 16


def _emb_body(hour_hbm, dow_hbm, mt_hbm, out_hbm,
              hour_v, dow_v, mt_v, out_v, sem):
    wid = lax.axis_index("s") * NC + lax.axis_index("c")
    base = wid * BPW
    copies = [
        pltpu.async_copy(mt_hbm, mt_v, sem),
        pltpu.async_copy(hour_hbm.at[pl.ds(base, BPW)], hour_v, sem),
        pltpu.async_copy(dow_hbm.at[pl.ds(base, BPW)], dow_v, sem),
    ]
    for c in copies:
        c.wait()
    for g in range(BPW // L):
        sl = pl.ds(g * L, L)
        hb = hour_v[sl] * SH
        db = dow_v[sl] * SD + DBASE
        for c in range(DH):
            out_v[c, sl] = plsc.load_gather(mt_v, [hb + c])
        for c in range(DD):
            out_v[DH + c, sl] = plsc.load_gather(mt_v, [db + c])
    pltpu.sync_copy(out_v, out_hbm.at[:, pl.ds(base, BPW)])


@jax.jit
def _lookup(hour, dow, mt):
    mesh = plsc.VectorSubcoreMesh(core_axis_name="c", subcore_axis_name="s")
    f = pl.kernel(
        _emb_body,
        out_type=jax.ShapeDtypeStruct((D, B), jnp.float32),
        mesh=mesh,
        compiler_params=pltpu.CompilerParams(
            needs_layout_passes=False, use_tc_tiling_on_sc=False),
        scratch_types=[
            pltpu.VMEM((BPW,), jnp.int32),
            pltpu.VMEM((BPW,), jnp.int32),
            pltpu.VMEM((MTP,), jnp.float32),
            pltpu.VMEM((D, BPW), jnp.float32),
            pltpu.SemaphoreType.DMA,
        ],
    )
    return f(hour, dow, mt)


def kernel(hour, dow, dom, hour_table, dow_table):
    del dom
    mt = jnp.concatenate([
        jnp.pad(hour_table, ((0, 0), (0, SH - DH))).reshape(-1),
        jnp.pad(dow_table, ((0, 0), (0, SD - DD))).reshape(-1),
        jnp.zeros((MTP - MT,), jnp.float32),
    ])
    out_t = _lookup(hour.astype(jnp.int32), dow.astype(jnp.int32), mt)
    return out_t.T


# speedup vs baseline: 5.6826x; 1.0006x over previous
import jax
import jax.numpy as jnp
from jax import lax
from jax.experimental import pallas as pl
from jax.experimental.pallas import tpu as pltpu
from jax.experimental.pallas import tpu_sc as plsc

B = 16384
NH = 24
DH = 8
ND = 7
DD = 4
D = DH + DD
NC = 2
NS = 16
NW = NC * NS
BPW = B // NW
L = 16


def _emb_body(hour_hbm, dow_hbm, ht_hbm, dt_hbm, out_hbm,
              hour_v, dow_v, ht_v, dt_v, out_v, sem):
    wid = lax.axis_index("s") * NC + lax.axis_index("c")
    base = wid * BPW
    copies = [
        pltpu.async_copy(ht_hbm, ht_v, sem),
        pltpu.async_copy(dt_hbm, dt_v, sem),
        pltpu.async_copy(hour_hbm.at[pl.ds(base, BPW)], hour_v, sem),
        pltpu.async_copy(dow_hbm.at[pl.ds(base, BPW)], dow_v, sem),
    ]
    for c in copies:
        c.wait()
    cvec = [jnp.full((L,), c, jnp.int32) for c in range(DH)]
    for g in range(BPW // L):
        sl = pl.ds(g * L, L)
        hv = hour_v[sl]
        dv = dow_v[sl]
        for c in range(DH):
            out_v[c, sl] = plsc.load_gather(ht_v, [cvec[c], hv])
        for c in range(DD):
            out_v[DH + c, sl] = plsc.load_gather(dt_v, [cvec[c], dv])
    pltpu.sync_copy(out_v, out_hbm.at[:, pl.ds(base, BPW)])


@jax.jit
def _lookup(hour, dow, ht_t, dt_t):
    mesh = plsc.VectorSubcoreMesh(core_axis_name="c", subcore_axis_name="s")
    f = pl.kernel(
        _emb_body,
        out_type=jax.ShapeDtypeStruct((D, B), jnp.float32),
        mesh=mesh,
        compiler_params=pltpu.CompilerParams(
            needs_layout_passes=False, use_tc_tiling_on_sc=False),
        scratch_types=[
            pltpu.VMEM((BPW,), jnp.int32),
            pltpu.VMEM((BPW,), jnp.int32),
            pltpu.VMEM((DH, NH), jnp.float32),
            pltpu.VMEM((DD, ND), jnp.float32),
            pltpu.VMEM((D, BPW), jnp.float32),
            pltpu.SemaphoreType.DMA,
        ],
    )
    return f(hour, dow, ht_t, dt_t)


def kernel(hour, dow, dom, hour_table, dow_table):
    del dom
    out_t = _lookup(hour.astype(jnp.int32), dow.astype(jnp.int32),
                    hour_table.T, dow_table.T)
    return out_t.T


# speedup vs baseline: 5.6912x; 1.0015x over previous
import jax
import jax.numpy as jnp
from jax import lax
from jax.experimental import pallas as pl
from jax.experimental.pallas import tpu as pltpu
from jax.experimental.pallas import tpu_sc as plsc

B = 16384
NH = 24
DH = 8
ND = 7
DD = 4
D = DH + DD
NC = 2
NS = 16
NW = NC * NS
BPW = B // NW
L = 16


def _emb_body(hour_hbm, dow_hbm, ht_hbm, dt_hbm, out_hbm,
              hour_v, dow_v, ht_v, dt_v, out_v, sem):
    wid = lax.axis_index("s") * NC + lax.axis_index("c")
    base = wid * BPW
    copies = [
        pltpu.async_copy(ht_hbm, ht_v, sem),
        pltpu.async_copy(dt_hbm, dt_v, sem),
        pltpu.async_copy(hour_hbm.at[pl.ds(base, BPW)], hour_v, sem),
        pltpu.async_copy(dow_hbm.at[pl.ds(base, BPW)], dow_v, sem),
    ]
    for c in copies:
        c.wait()
    cvec = [jnp.full((L,), c, jnp.int32) for c in range(DH)]
    for g in range(BPW // L):
        sl = pl.ds(g * L, L)
        hv = hour_v[sl]
        dv = dow_v[sl]
        for c in range(DH):
            out_v[c, sl] = plsc.load_gather(ht_v, [cvec[c], hv])
        for c in range(DD):
            out_v[DH + c, sl] = plsc.load_gather(dt_v, [cvec[c], dv])
    pltpu.sync_copy(out_v, out_hbm.at[:, pl.ds(base, BPW)])


@jax.jit
def _lookup(hour, dow, ht_t, dt_t):
    mesh = plsc.VectorSubcoreMesh(core_axis_name="c", subcore_axis_name="s")
    f = pl.kernel(
        _emb_body,
        out_type=jax.ShapeDtypeStruct((D, B), jnp.float32),
        mesh=mesh,
        compiler_params=pltpu.CompilerParams(
            needs_layout_passes=False, use_tc_tiling_on_sc=False,
            skip_device_barrier=True, disable_bounds_checks=True,
            disable_semaphore_checks=True),
        scratch_types=[
            pltpu.VMEM((BPW,), jnp.int32),
            pltpu.VMEM((BPW,), jnp.int32),
            pltpu.VMEM((DH, NH), jnp.float32),
            pltpu.VMEM((DD, ND), jnp.float32),
            pltpu.VMEM((D, BPW), jnp.float32),
            pltpu.SemaphoreType.DMA,
        ],
    )
    return f(hour, dow, ht_t, dt_t)


def kernel(hour, dow, dom, hour_table, dow_table):
    del dom
    out_t = _lookup(hour.astype(jnp.int32), dow.astype(jnp.int32),
                    hour_table.T, dow_table.T)
    return out_t.T


# speedup vs baseline: 6.1146x; 1.0744x over previous
import jax
import jax.numpy as jnp
from jax import lax
from jax.experimental import pallas as pl
from jax.experimental.pallas import tpu as pltpu
from jax.experimental.pallas import tpu_sc as plsc

B = 16384
NH = 24
DH = 8
ND = 7
DD = 4
D = DH + DD
NC = 2
NS = 16
NW = NC * NS
BPW = B // NW
L = 16


def _emb_body(hour_hbm, dow_hbm, ht_hbm, dt_hbm, out_hbm,
              hour_v, dow_v, ht_v, dt_v, out_v, sem):
    wid = lax.axis_index("s") * NC + lax.axis_index("c")
    base = wid * BPW
    copies = [
        pltpu.async_copy(ht_hbm, ht_v, sem),
        pltpu.async_copy(dt_hbm, dt_v, sem),
        pltpu.async_copy(hour_hbm.at[pl.ds(base, BPW)], hour_v, sem),
        pltpu.async_copy(dow_hbm.at[pl.ds(base, BPW)], dow_v, sem),
    ]
    for c in copies:
        c.wait()
    cvec = [jnp.full((L,), c, jnp.int32) for c in range(DH)]
    for g in range(BPW // L):
        sl = pl.ds(g * L, L)
        hv = hour_v[sl]
        dv = dow_v[sl]
        for c in range(DH):
            out_v[c, sl] = plsc.load_gather(ht_v, [cvec[c], hv])
        for c in range(DD):
            out_v[DH + c, sl] = plsc.load_gather(dt_v, [cvec[c], dv])
    pltpu.sync_copy(out_v, out_hbm.at[:, pl.ds(base, BPW)])


@jax.jit
def _lookup(hour, dow, ht_t, dt_t):
    mesh = plsc.VectorSubcoreMesh(core_axis_name="c", subcore_axis_name="s")
    f = pl.kernel(
        _emb_body,
        out_type=jax.ShapeDtypeStruct((D, B), jnp.float32),
        mesh=mesh,
        compiler_params=pltpu.CompilerParams(needs_layout_passes=False),
        scratch_types=[
            pltpu.VMEM((BPW,), jnp.int32),
            pltpu.VMEM((BPW,), jnp.int32),
            pltpu.VMEM((DH, NH), jnp.float32),
            pltpu.VMEM((DD, ND), jnp.float32),
            pltpu.VMEM((D, BPW), jnp.float32),
            pltpu.SemaphoreType.DMA,
        ],
    )
    return f(hour, dow, ht_t, dt_t)


def kernel(hour, dow, dom, hour_table, dow_table):
    del dom
    out_t = _lookup(hour.astype(jnp.int32), dow.astype(jnp.int32),
                    hour_table.T, dow_table.T)
    return out_t.T
